# baseline (speedup 1.0000x reference)
import dataclasses
import functools

import jax
import jax.numpy as jnp
from jax import lax
from jax.experimental import pallas as pl
from jax.experimental.pallas import tpu as pltpu
from jax.experimental.pallas import tpu_sc as plsc

_N = 10000
_E = 320000
_D = 128
_G = 64
_NC, _NS = 2, 16
_NW = _NC * _NS
_NPAD = 10240
_EPW = _E // _NW
_K = 80
_CPW = _EPW // _K
_BLK = 1024
_NB = _NPAD // _BLK
_HI = lax.Precision.HIGHEST

def _mesh():
    return plsc.VectorSubcoreMesh(core_axis_name="c", subcore_axis_name="s",
                                  num_cores=_NC, num_subcores=_NS)


def _sc_params():
    cp = pltpu.CompilerParams()
    if "needs_layout_passes" in pltpu.CompilerParams.__dataclass_fields__:
        cp = dataclasses.replace(cp, needs_layout_passes=False)
    return cp


def _sc_degree(dst):
    @functools.partial(
        pl.kernel,
        out_type=jax.ShapeDtypeStruct((_NW, _NPAD), jnp.float32),
        mesh=_mesh(),
        scratch_types=[
            pltpu.VMEM((_EPW,), jnp.int32),
            pltpu.VMEM((_NPAD,), jnp.float32),
        ],
        compiler_params=_sc_params(),
    )
    def k(dst_hbm, out_hbm, dst_v, deg_v):
        c = lax.axis_index("c")
        s = lax.axis_index("s")
        wid = s * _NC + c
        zero16 = jnp.zeros((16,), jnp.float32)
        ones16 = jnp.ones((16,), jnp.float32)

        @pl.loop(0, _NPAD, step=16)
        def _(i):
            deg_v[pl.ds(i, 16)] = zero16

        pltpu.sync_copy(dst_hbm.at[pl.ds(wid * _EPW, _EPW)], dst_v)

        @pl.loop(0, _EPW, step=16)
        def _(i):
            idx = dst_v[pl.ds(i, 16)]
            plsc.addupdate_scatter(deg_v, [idx], ones16)

        pltpu.sync_copy(deg_v, out_hbm.at[wid])

    return k(dst)


_RPW = 80
_ROWS = _RPW * _NW
_EPAD = _ROWS * 128 - _E


def _sc_aggregate(y, src2d, dst2d, z):
    @functools.partial(
        pl.kernel,
        out_type=jax.ShapeDtypeStruct((_NC, _NPAD, _D), jnp.float32),
        mesh=_mesh(),
        scratch_types=[
            pltpu.VMEM((_RPW // 2, 128), jnp.int32),
            pltpu.VMEM((_RPW // 2, 128), jnp.int32),
            pltpu.VMEM((128, _D), jnp.float32),
            pltpu.VMEM((128, _D), jnp.float32),
            pltpu.VMEM_SHARED((_NPAD, _D), jnp.float32),
            pltpu.SemaphoreType.DMA,
            pltpu.SemaphoreType.DMA,
            pltpu.SemaphoreType.DMA,
            pltpu.SemaphoreType.DMA,
        ],
    )
    def k(y_hbm, src_hbm, dst_hbm, z_hbm, out_hbm, src_v, dst_v, b0, b1,
          acc_sh, gs0, gs1, ss0, ss1):
        c = lax.axis_index("c")
        s = lax.axis_index("s")
        wid = s * _NC + c
        rps = _NPAD // _NS
        base = wid * _RPW
        half = _RPW // 2

        pltpu.sync_copy(z_hbm, acc_sh.at[pl.ds(s * rps, rps)])
        plsc.subcore_barrier()

        def g_start(j, buf, sem):
            pltpu.async_copy(y_hbm.at[src_v.at[j]], buf, sem)

        def g_wait(j, buf, sem):
            pltpu.make_async_copy(y_hbm.at[src_v.at[j]], buf, sem).wait()

        def s_start(j, buf, sem):
            pltpu.async_copy(buf, acc_sh.at[dst_v.at[j]], sem, add=True)

        def s_wait(j, buf, sem):
            pltpu.make_async_copy(buf, acc_sh.at[dst_v.at[j]], sem).wait()

        @pl.loop(0, 2)
        def _(p):
            pltpu.sync_copy(src_hbm.at[pl.ds(base + p * half, half)], src_v)
            pltpu.sync_copy(dst_hbm.at[pl.ds(base + p * half, half)], dst_v)

            g_start(0, b0, gs0)
            g_start(1, b1, gs1)

            @pl.loop(0, half - 2, step=2)
            def _(j):
                g_wait(j, b0, gs0)
                s_start(j, b0, ss0)
                g_wait(j + 1, b1, gs1)
                s_start(j + 1, b1, ss1)
                s_wait(j, b0, ss0)
                g_start(j + 2, b0, gs0)
                s_wait(j + 1, b1, ss1)
                g_start(j + 3, b1, gs1)

            g_wait(half - 2, b0, gs0)
            s_start(half - 2, b0, ss0)
            g_wait(half - 1, b1, gs1)
            s_start(half - 1, b1, ss1)
            s_wait(half - 2, b0, ss0)
            s_wait(half - 1, b1, ss1)

        plsc.subcore_barrier()
        pltpu.sync_copy(acc_sh.at[pl.ds(s * rps, rps)],
                        out_hbm.at[c, pl.ds(s * rps, rps)])

    return k(y, src2d, dst2d, z)


def _dis_of(degp_blk):
    deg = jnp.sum(degp_blk, axis=0) + 1.0
    return lax.rsqrt(deg)


def _mm_scale_body(x_ref, w_ref, degp_ref, o_ref):
    xw = jnp.dot(x_ref[...], w_ref[...], precision=_HI,
                 preferred_element_type=jnp.float32)
    dis = _dis_of(degp_ref[...])
    o_ref[...] = xw * dis[:, None]


def _mm_scale(x, w, degp):
    return pl.pallas_call(
        _mm_scale_body,
        grid=(_NB,),
        in_specs=[
            pl.BlockSpec((_BLK, _D), lambda i: (i, 0)),
            pl.BlockSpec((_D, _D), lambda i: (0, 0)),
            pl.BlockSpec((_NW, _BLK), lambda i: (0, i)),
        ],
        out_specs=pl.BlockSpec((_BLK, _D), lambda i: (i, 0)),
        out_shape=jax.ShapeDtypeStruct((_NPAD, _D), jnp.float32),
    )(x, w, degp)


def _layer2_body(p_ref, y_ref, degp_ref, b1_ref, w2_ref, o_ref):
    dis = _dis_of(degp_ref[...])
    h = dis[:, None] * (p_ref[0] + p_ref[1] + y_ref[...]) + b1_ref[...]
    h = jnp.maximum(h, 0.0)
    y2 = jnp.dot(h, w2_ref[...], precision=_HI,
                 preferred_element_type=jnp.float32)
    o_ref[...] = y2 * dis[:, None]


def _layer2(p, y1, degp, b1, w2):
    return pl.pallas_call(
        _layer2_body,
        grid=(_NB,),
        in_specs=[
            pl.BlockSpec((_NC, _BLK, _D), lambda i: (0, i, 0)),
            pl.BlockSpec((_BLK, _D), lambda i: (i, 0)),
            pl.BlockSpec((_NW, _BLK), lambda i: (0, i)),
            pl.BlockSpec((1, _D), lambda i: (0, 0)),
            pl.BlockSpec((_D, _D), lambda i: (0, 0)),
        ],
        out_specs=pl.BlockSpec((_BLK, _D), lambda i: (i, 0)),
        out_shape=jax.ShapeDtypeStruct((_NPAD, _D), jnp.float32),
    )(p, y1, degp, b1, w2)


def _pool_body(q_ref, y2_ref, degp_ref, batch_ref, b2_ref, o_ref, cnt_ref):
    i = pl.program_id(0)
    dis = _dis_of(degp_ref[...])
    nodes = dis[:, None] * (q_ref[0] + q_ref[1] + y2_ref[...])
    gids = lax.broadcasted_iota(jnp.int32, (_BLK, _G), 1)
    onehot = (batch_ref[0, 0, :][:, None] == gids).astype(jnp.float32)
    sums = lax.dot_general(onehot, nodes, (((0,), (0,)), ((), ())),
                           precision=_HI,
                           preferred_element_type=jnp.float32)
    cnts = jnp.sum(onehot, axis=0)

    @pl.when(i == 0)
    def _():
        o_ref[...] = jnp.zeros_like(o_ref)
        cnt_ref[...] = jnp.zeros_like(cnt_ref)

    o_ref[...] += sums
    cnt_ref[...] += cnts[:, None]

    @pl.when(i == _NB - 1)
    def _():
        o_ref[...] = o_ref[...] / jnp.maximum(cnt_ref[...], 1.0) + b2_ref[...]


def _pool(q, y2, degp, batch3, b2):
    return pl.pallas_call(
        _pool_body,
        grid=(_NB,),
        in_specs=[
            pl.BlockSpec((_NC, _BLK, _D), lambda i: (0, i, 0)),
            pl.BlockSpec((_BLK, _D), lambda i: (i, 0)),
            pl.BlockSpec((_NW, _BLK), lambda i: (0, i)),
            pl.BlockSpec((1, 1, _BLK), lambda i: (i, 0, 0)),
            pl.BlockSpec((1, _D), lambda i: (0, 0)),
        ],
        out_specs=pl.BlockSpec((_G, _D), lambda i: (0, 0)),
        out_shape=jax.ShapeDtypeStruct((_G, _D), jnp.float32),
        scratch_shapes=[pltpu.VMEM((_G, _D), jnp.float32)],
    )(q, y2, degp, batch3, b2)


def kernel(x, edge_index, batch, W1, b1, W2, b2):
    src = edge_index[0]
    dst = edge_index[1]
    pad_i = jnp.arange(_EPAD, dtype=jnp.int32)
    src2d = jnp.concatenate(
        [src, pad_i % _N]).reshape(_ROWS, 128)
    dst2d = jnp.concatenate(
        [dst, _N + pad_i % (_NPAD - _N)]).reshape(_ROWS, 128)
    z = jnp.zeros((_NPAD // _NS, _D), jnp.float32)
    b1r = b1.reshape(1, _D)
    b2r = b2.reshape(1, _D)
    xp = jnp.pad(x, ((0, _NPAD - _N), (0, 0)))
    batch3 = jnp.pad(batch, (0, _NPAD - _N),
                     constant_values=_G).reshape(_NB, 1, _BLK)

    degp = _sc_degree(dst)
    y1 = _mm_scale(xp, W1, degp)
    p = _sc_aggregate(y1, src2d, dst2d, z)
    y2 = _layer2(p, y1, degp, b1r, W2)
    q = _sc_aggregate(y2, src2d, dst2d, z)
    return _pool(q, y2, degp, batch3, b2r)


# speedup vs baseline: 1.2421x; 1.2421x over previous
import dataclasses
import functools

import jax
import jax.numpy as jnp
from jax import lax
from jax.experimental import pallas as pl
from jax.experimental.pallas import tpu as pltpu
from jax.experimental.pallas import tpu_sc as plsc

_N = 10000
_E = 320000
_D = 128
_G = 64
_NC, _NS = 2, 16
_NW = _NC * _NS
_NPAD = 10240
_EPW = _E // _NW
_K = 80
_CPW = _EPW // _K
_BLK = 1024
_NB = _NPAD // _BLK
_HI = lax.Precision.HIGHEST

def _mesh():
    return plsc.VectorSubcoreMesh(core_axis_name="c", subcore_axis_name="s",
                                  num_cores=_NC, num_subcores=_NS)


def _sc_params():
    cp = pltpu.CompilerParams()
    if "needs_layout_passes" in pltpu.CompilerParams.__dataclass_fields__:
        cp = dataclasses.replace(cp, needs_layout_passes=False)
    return cp


def _sc_degree(dst):
    @functools.partial(
        pl.kernel,
        out_type=jax.ShapeDtypeStruct((_NW, _NPAD), jnp.float32),
        mesh=_mesh(),
        scratch_types=[
            pltpu.VMEM((_EPW,), jnp.int32),
            pltpu.VMEM((_NPAD,), jnp.float32),
        ],
        compiler_params=_sc_params(),
    )
    def k(dst_hbm, out_hbm, dst_v, deg_v):
        c = lax.axis_index("c")
        s = lax.axis_index("s")
        wid = s * _NC + c
        zero16 = jnp.zeros((16,), jnp.float32)
        ones16 = jnp.ones((16,), jnp.float32)

        @pl.loop(0, _NPAD, step=16)
        def _(i):
            deg_v[pl.ds(i, 16)] = zero16

        pltpu.sync_copy(dst_hbm.at[pl.ds(wid * _EPW, _EPW)], dst_v)

        @pl.loop(0, _EPW, step=16)
        def _(i):
            idx = dst_v[pl.ds(i, 16)]
            plsc.addupdate_scatter(deg_v, [idx], ones16)

        pltpu.sync_copy(deg_v, out_hbm.at[wid])

    return k(dst)


_RPW = 80
_ROWS = _RPW * _NW
_EPAD = _ROWS * 128 - _E


def _sc_aggregate(y, src2d, dst2d, z):
    @functools.partial(
        pl.kernel,
        out_type=jax.ShapeDtypeStruct((_NC, _NPAD, _D), jnp.float32),
        mesh=_mesh(),
        scratch_types=[
            pltpu.VMEM((_RPW // 2, 128), jnp.int32),
            pltpu.VMEM((_RPW // 2, 128), jnp.int32),
            pltpu.VMEM((128, _D), jnp.float32),
            pltpu.VMEM((128, _D), jnp.float32),
            pltpu.VMEM_SHARED((_NPAD, _D), jnp.float32),
            pltpu.SemaphoreType.DMA,
            pltpu.SemaphoreType.DMA,
            pltpu.SemaphoreType.DMA,
            pltpu.SemaphoreType.DMA,
        ],
    )
    def k(y_hbm, src_hbm, dst_hbm, z_hbm, out_hbm, src_v, dst_v, b0, b1,
          acc_sh, gs0, gs1, ss0, ss1):
        c = lax.axis_index("c")
        s = lax.axis_index("s")
        wid = s * _NC + c
        rps = _NPAD // _NS
        base = wid * _RPW
        half = _RPW // 2

        pltpu.sync_copy(z_hbm, acc_sh.at[pl.ds(s * rps, rps)])
        plsc.subcore_barrier()

        def g_start(j, buf, sem):
            pltpu.async_copy(y_hbm.at[src_v.at[j]], buf, sem)

        def g_wait(j, buf, sem):
            pltpu.make_async_copy(y_hbm.at[src_v.at[j]], buf, sem).wait()

        def scat(j, buf):
            pltpu.sync_copy(buf, acc_sh.at[dst_v.at[j]], add=True)

        @pl.loop(0, 2)
        def _(p):
            pltpu.sync_copy(src_hbm.at[pl.ds(base + p * half, half)], src_v)
            pltpu.sync_copy(dst_hbm.at[pl.ds(base + p * half, half)], dst_v)

            g_start(0, b0, gs0)

            @pl.loop(0, half - 2, step=2)
            def _(j):
                g_start(j + 1, b1, gs1)
                g_wait(j, b0, gs0)
                scat(j, b0)
                g_start(j + 2, b0, gs0)
                g_wait(j + 1, b1, gs1)
                scat(j + 1, b1)

            g_start(half - 1, b1, gs1)
            g_wait(half - 2, b0, gs0)
            scat(half - 2, b0)
            g_wait(half - 1, b1, gs1)
            scat(half - 1, b1)

        plsc.subcore_barrier()
        pltpu.sync_copy(acc_sh.at[pl.ds(s * rps, rps)],
                        out_hbm.at[c, pl.ds(s * rps, rps)])

    return k(y, src2d, dst2d, z)


def _dis_of(degp_blk):
    deg = jnp.sum(degp_blk, axis=0) + 1.0
    return lax.rsqrt(deg)


def _mm_scale_body(x_ref, w_ref, degp_ref, o_ref):
    xw = jnp.dot(x_ref[...], w_ref[...], precision=_HI,
                 preferred_element_type=jnp.float32)
    dis = _dis_of(degp_ref[...])
    o_ref[...] = xw * dis[:, None]


def _mm_scale(x, w, degp):
    return pl.pallas_call(
        _mm_scale_body,
        grid=(_NB,),
        in_specs=[
            pl.BlockSpec((_BLK, _D), lambda i: (i, 0)),
            pl.BlockSpec((_D, _D), lambda i: (0, 0)),
            pl.BlockSpec((_NW, _BLK), lambda i: (0, i)),
        ],
        out_specs=pl.BlockSpec((_BLK, _D), lambda i: (i, 0)),
        out_shape=jax.ShapeDtypeStruct((_NPAD, _D), jnp.float32),
    )(x, w, degp)


def _layer2_body(p_ref, y_ref, degp_ref, b1_ref, w2_ref, o_ref):
    dis = _dis_of(degp_ref[...])
    h = dis[:, None] * (p_ref[0] + p_ref[1] + y_ref[...]) + b1_ref[...]
    h = jnp.maximum(h, 0.0)
    y2 = jnp.dot(h, w2_ref[...], precision=_HI,
                 preferred_element_type=jnp.float32)
    o_ref[...] = y2 * dis[:, None]


def _layer2(p, y1, degp, b1, w2):
    return pl.pallas_call(
        _layer2_body,
        grid=(_NB,),
        in_specs=[
            pl.BlockSpec((_NC, _BLK, _D), lambda i: (0, i, 0)),
            pl.BlockSpec((_BLK, _D), lambda i: (i, 0)),
            pl.BlockSpec((_NW, _BLK), lambda i: (0, i)),
            pl.BlockSpec((1, _D), lambda i: (0, 0)),
            pl.BlockSpec((_D, _D), lambda i: (0, 0)),
        ],
        out_specs=pl.BlockSpec((_BLK, _D), lambda i: (i, 0)),
        out_shape=jax.ShapeDtypeStruct((_NPAD, _D), jnp.float32),
    )(p, y1, degp, b1, w2)


def _pool_body(q_ref, y2_ref, degp_ref, batch_ref, b2_ref, o_ref, cnt_ref):
    i = pl.program_id(0)
    dis = _dis_of(degp_ref[...])
    nodes = dis[:, None] * (q_ref[0] + q_ref[1] + y2_ref[...])
    gids = lax.broadcasted_iota(jnp.int32, (_BLK, _G), 1)
    onehot = (batch_ref[0, 0, :][:, None] == gids).astype(jnp.float32)
    sums = lax.dot_general(onehot, nodes, (((0,), (0,)), ((), ())),
                           precision=_HI,
                           preferred_element_type=jnp.float32)
    cnts = jnp.sum(onehot, axis=0)

    @pl.when(i == 0)
    def _():
        o_ref[...] = jnp.zeros_like(o_ref)
        cnt_ref[...] = jnp.zeros_like(cnt_ref)

    o_ref[...] += sums
    cnt_ref[...] += cnts[:, None]

    @pl.when(i == _NB - 1)
    def _():
        o_ref[...] = o_ref[...] / jnp.maximum(cnt_ref[...], 1.0) + b2_ref[...]


def _pool(q, y2, degp, batch3, b2):
    return pl.pallas_call(
        _pool_body,
        grid=(_NB,),
        in_specs=[
            pl.BlockSpec((_NC, _BLK, _D), lambda i: (0, i, 0)),
            pl.BlockSpec((_BLK, _D), lambda i: (i, 0)),
            pl.BlockSpec((_NW, _BLK), lambda i: (0, i)),
            pl.BlockSpec((1, 1, _BLK), lambda i: (i, 0, 0)),
            pl.BlockSpec((1, _D), lambda i: (0, 0)),
        ],
        out_specs=pl.BlockSpec((_G, _D), lambda i: (0, 0)),
        out_shape=jax.ShapeDtypeStruct((_G, _D), jnp.float32),
        scratch_shapes=[pltpu.VMEM((_G, _D), jnp.float32)],
    )(q, y2, degp, batch3, b2)


def kernel(x, edge_index, batch, W1, b1, W2, b2):
    src = edge_index[0]
    dst = edge_index[1]
    pad_i = jnp.arange(_EPAD, dtype=jnp.int32)
    src2d = jnp.concatenate(
        [src, pad_i % _N]).reshape(_ROWS, 128)
    dst2d = jnp.concatenate(
        [dst, _N + pad_i % (_NPAD - _N)]).reshape(_ROWS, 128)
    z = jnp.zeros((_NPAD // _NS, _D), jnp.float32)
    b1r = b1.reshape(1, _D)
    b2r = b2.reshape(1, _D)
    xp = jnp.pad(x, ((0, _NPAD - _N), (0, 0)))
    batch3 = jnp.pad(batch, (0, _NPAD - _N),
                     constant_values=_G).reshape(_NB, 1, _BLK)

    degp = _sc_degree(dst)
    y1 = _mm_scale(xp, W1, degp)
    p = _sc_aggregate(y1, src2d, dst2d, z)
    y2 = _layer2(p, y1, degp, b1r, W2)
    q = _sc_aggregate(y2, src2d, dst2d, z)
    return _pool(q, y2, degp, batch3, b2r)


# speedup vs baseline: 1.2784x; 1.0292x over previous
import dataclasses
import functools

import jax
import jax.numpy as jnp
from jax import lax
from jax.experimental import pallas as pl
from jax.experimental.pallas import tpu as pltpu
from jax.experimental.pallas import tpu_sc as plsc

_N = 10000
_E = 320000
_D = 128
_G = 64
_NC, _NS = 2, 16
_NW = _NC * _NS
_NPAD = 10240
_EPW = _E // _NW
_K = 80
_CPW = _EPW // _K
_BLK = 1024
_NB = _NPAD // _BLK
_HI = lax.Precision.HIGHEST

def _mesh():
    return plsc.VectorSubcoreMesh(core_axis_name="c", subcore_axis_name="s",
                                  num_cores=_NC, num_subcores=_NS)


def _sc_params():
    cp = pltpu.CompilerParams()
    if "needs_layout_passes" in pltpu.CompilerParams.__dataclass_fields__:
        cp = dataclasses.replace(cp, needs_layout_passes=False)
    return cp


def _sc_degree(ei3, tail3):
    @functools.partial(
        pl.kernel,
        out_type=jax.ShapeDtypeStruct((_NW, _NPAD), jnp.float32),
        mesh=_mesh(),
        scratch_types=[
            pltpu.VMEM((_RPW, 128), jnp.int32),
            pltpu.VMEM((_NPAD,), jnp.float32),
        ],
        compiler_params=_sc_params(),
    )
    def k(ei_hbm, tail_hbm, out_hbm, dst_v, deg_v):
        c = lax.axis_index("c")
        s = lax.axis_index("s")
        wid = s * _NC + c
        zero16 = jnp.zeros((16,), jnp.float32)
        ones16 = jnp.ones((16,), jnp.float32)

        @pl.loop(0, _NPAD, step=16)
        def _(i):
            deg_v[pl.ds(i, 16)] = zero16

        def count_rows(n):
            @pl.loop(0, n)
            def _(r):
                for kk in range(128 // 16):
                    idx = dst_v[r, pl.ds(kk * 16, 16)]
                    plsc.addupdate_scatter(deg_v, [idx], ones16)

        @pl.when(wid < _NW - 1)
        def _():
            pltpu.sync_copy(ei_hbm.at[1, pl.ds(wid * _RPW, _RPW)], dst_v)
            count_rows(_RPW)

        @pl.when(wid == _NW - 1)
        def _():
            pltpu.sync_copy(tail_hbm.at[1], dst_v.at[pl.ds(0, _TAIL)])
            count_rows(_TAIL)

        pltpu.sync_copy(deg_v, out_hbm.at[wid])

    return k(ei3, tail3)


_RPW = 80
_ROWS = _E // 128
_TAIL = _ROWS - 31 * _RPW


def _sc_aggregate(y, ei3, tail3, z):
    @functools.partial(
        pl.kernel,
        out_type=jax.ShapeDtypeStruct((_NC, _NPAD, _D), jnp.float32),
        mesh=_mesh(),
        scratch_types=[
            pltpu.VMEM((_RPW // 2, 128), jnp.int32),
            pltpu.VMEM((_RPW // 2, 128), jnp.int32),
            pltpu.VMEM((128, _D), jnp.float32),
            pltpu.VMEM((128, _D), jnp.float32),
            pltpu.VMEM_SHARED((_NPAD, _D), jnp.float32),
            pltpu.SemaphoreType.DMA,
            pltpu.SemaphoreType.DMA,
            pltpu.SemaphoreType.DMA,
            pltpu.SemaphoreType.DMA,
        ],
    )
    def k(y_hbm, ei_hbm, tail_hbm, z_hbm, out_hbm, src_v, dst_v, b0, b1,
          acc_sh, gs0, gs1, ss0, ss1):
        c = lax.axis_index("c")
        s = lax.axis_index("s")
        wid = s * _NC + c
        rps = _NPAD // _NS
        base = wid * _RPW
        half = _RPW // 2

        pltpu.sync_copy(z_hbm, acc_sh.at[pl.ds(s * rps, rps)])
        plsc.subcore_barrier()

        def g_start(j, buf, sem):
            pltpu.async_copy(y_hbm.at[src_v.at[j]], buf, sem)

        def g_wait(j, buf, sem):
            pltpu.make_async_copy(y_hbm.at[src_v.at[j]], buf, sem).wait()

        def scat(j, buf):
            pltpu.sync_copy(buf, acc_sh.at[dst_v.at[j]], add=True)

        def pipeline(n):
            g_start(0, b0, gs0)

            @pl.loop(0, n - 2, step=2)
            def _(j):
                g_start(j + 1, b1, gs1)
                g_wait(j, b0, gs0)
                scat(j, b0)
                g_start(j + 2, b0, gs0)
                g_wait(j + 1, b1, gs1)
                scat(j + 1, b1)

            g_start(n - 1, b1, gs1)
            g_wait(n - 2, b0, gs0)
            scat(n - 2, b0)
            g_wait(n - 1, b1, gs1)
            scat(n - 1, b1)

        @pl.when(wid < _NW - 1)
        def _():
            @pl.loop(0, 2)
            def _(p):
                row0 = base + p * half
                pltpu.sync_copy(ei_hbm.at[0, pl.ds(row0, half)], src_v)
                pltpu.sync_copy(ei_hbm.at[1, pl.ds(row0, half)], dst_v)
                pipeline(half)

        @pl.when(wid == _NW - 1)
        def _():
            pltpu.sync_copy(tail_hbm.at[0], src_v.at[pl.ds(0, _TAIL)])
            pltpu.sync_copy(tail_hbm.at[1], dst_v.at[pl.ds(0, _TAIL)])
            pipeline(_TAIL)

        plsc.subcore_barrier()
        pltpu.sync_copy(acc_sh.at[pl.ds(s * rps, rps)],
                        out_hbm.at[c, pl.ds(s * rps, rps)])

    return k(y, ei3, tail3, z)


def _dis_of(degp_blk):
    deg = jnp.sum(degp_blk, axis=0) + 1.0
    return lax.rsqrt(deg)


def _mm_scale_body(x_ref, w_ref, degp_ref, o_ref):
    xw = jnp.dot(x_ref[...], w_ref[...], precision=_HI,
                 preferred_element_type=jnp.float32)
    dis = _dis_of(degp_ref[...])
    o_ref[...] = xw * dis[:, None]


def _mm_scale(x, w, degp):
    return pl.pallas_call(
        _mm_scale_body,
        grid=(_NB,),
        in_specs=[
            pl.BlockSpec((_BLK, _D), lambda i: (i, 0)),
            pl.BlockSpec((_D, _D), lambda i: (0, 0)),
            pl.BlockSpec((_NW, _BLK), lambda i: (0, i)),
        ],
        out_specs=pl.BlockSpec((_BLK, _D), lambda i: (i, 0)),
        out_shape=jax.ShapeDtypeStruct((_NPAD, _D), jnp.float32),
    )(x, w, degp)


def _layer2_body(p_ref, y_ref, degp_ref, b1_ref, w2_ref, o_ref):
    dis = _dis_of(degp_ref[...])
    h = dis[:, None] * (p_ref[0] + p_ref[1] + y_ref[...]) + b1_ref[...]
    h = jnp.maximum(h, 0.0)
    y2 = jnp.dot(h, w2_ref[...], precision=_HI,
                 preferred_element_type=jnp.float32)
    o_ref[...] = y2 * dis[:, None]


def _layer2(p, y1, degp, b1, w2):
    return pl.pallas_call(
        _layer2_body,
        grid=(_NB,),
        in_specs=[
            pl.BlockSpec((_NC, _BLK, _D), lambda i: (0, i, 0)),
            pl.BlockSpec((_BLK, _D), lambda i: (i, 0)),
            pl.BlockSpec((_NW, _BLK), lambda i: (0, i)),
            pl.BlockSpec((1, _D), lambda i: (0, 0)),
            pl.BlockSpec((_D, _D), lambda i: (0, 0)),
        ],
        out_specs=pl.BlockSpec((_BLK, _D), lambda i: (i, 0)),
        out_shape=jax.ShapeDtypeStruct((_NPAD, _D), jnp.float32),
    )(p, y1, degp, b1, w2)


def _pool_body(q_ref, y2_ref, degp_ref, batch_ref, b2_ref, o_ref, cnt_ref):
    i = pl.program_id(0)
    dis = _dis_of(degp_ref[...])
    nodes = dis[:, None] * (q_ref[0] + q_ref[1] + y2_ref[...])
    gids = lax.broadcasted_iota(jnp.int32, (_BLK, _G), 1)
    onehot = (batch_ref[0, 0, :][:, None] == gids).astype(jnp.float32)
    sums = lax.dot_general(onehot, nodes, (((0,), (0,)), ((), ())),
                           precision=_HI,
                           preferred_element_type=jnp.float32)
    cnts = jnp.sum(onehot, axis=0)

    @pl.when(i == 0)
    def _():
        o_ref[...] = jnp.zeros_like(o_ref)
        cnt_ref[...] = jnp.zeros_like(cnt_ref)

    o_ref[...] += sums
    cnt_ref[...] += cnts[:, None]

    @pl.when(i == _NB - 1)
    def _():
        o_ref[...] = o_ref[...] / jnp.maximum(cnt_ref[...], 1.0) + b2_ref[...]


def _pool(q, y2, degp, batch3, b2):
    return pl.pallas_call(
        _pool_body,
        grid=(_NB,),
        in_specs=[
            pl.BlockSpec((_NC, _BLK, _D), lambda i: (0, i, 0)),
            pl.BlockSpec((_BLK, _D), lambda i: (i, 0)),
            pl.BlockSpec((_NW, _BLK), lambda i: (0, i)),
            pl.BlockSpec((1, 1, _BLK), lambda i: (i, 0, 0)),
            pl.BlockSpec((1, _D), lambda i: (0, 0)),
        ],
        out_specs=pl.BlockSpec((_G, _D), lambda i: (0, 0)),
        out_shape=jax.ShapeDtypeStruct((_G, _D), jnp.float32),
        scratch_shapes=[pltpu.VMEM((_G, _D), jnp.float32)],
    )(q, y2, degp, batch3, b2)


def kernel(x, edge_index, batch, W1, b1, W2, b2):
    ei3 = edge_index.reshape(2, _ROWS, 128)
    tail3 = ei3[:, 31 * _RPW:, :]
    z = jnp.zeros((_NPAD // _NS, _D), jnp.float32)
    b1r = b1.reshape(1, _D)
    b2r = b2.reshape(1, _D)
    xp = jnp.pad(x, ((0, _NPAD - _N), (0, 0)))
    batch3 = jnp.pad(batch, (0, _NPAD - _N),
                     constant_values=_G).reshape(_NB, 1, _BLK)

    degp = _sc_degree(ei3, tail3)
    y1 = _mm_scale(xp, W1, degp)
    p = _sc_aggregate(y1, ei3, tail3, z)
    y2 = _layer2(p, y1, degp, b1r, W2)
    q = _sc_aggregate(y2, ei3, tail3, z)
    return _pool(q, y2, degp, batch3, b2r)


# speedup vs baseline: 1.3069x; 1.0223x over previous
import dataclasses
import functools

import jax
import jax.numpy as jnp
from jax import lax
from jax.experimental import pallas as pl
from jax.experimental.pallas import tpu as pltpu
from jax.experimental.pallas import tpu_sc as plsc

_N = 10000
_E = 320000
_D = 128
_G = 64
_NC, _NS = 2, 16
_NW = _NC * _NS
_NPAD = 10240
_EPW = _E // _NW
_K = 80
_CPW = _EPW // _K
_BLK = 2048
_NB = _NPAD // _BLK
_HI = lax.Precision.HIGHEST

def _mesh():
    return plsc.VectorSubcoreMesh(core_axis_name="c", subcore_axis_name="s",
                                  num_cores=_NC, num_subcores=_NS)


def _sc_params():
    cp = pltpu.CompilerParams()
    if "needs_layout_passes" in pltpu.CompilerParams.__dataclass_fields__:
        cp = dataclasses.replace(cp, needs_layout_passes=False)
    return cp


def _sc_degree(ei3, tail3):
    @functools.partial(
        pl.kernel,
        out_type=jax.ShapeDtypeStruct((_NW, _NPAD), jnp.float32),
        mesh=_mesh(),
        scratch_types=[
            pltpu.VMEM((_RPW, 128), jnp.int32),
            pltpu.VMEM((_NPAD,), jnp.float32),
        ],
        compiler_params=_sc_params(),
    )
    def k(ei_hbm, tail_hbm, out_hbm, dst_v, deg_v):
        c = lax.axis_index("c")
        s = lax.axis_index("s")
        wid = s * _NC + c
        zero16 = jnp.zeros((16,), jnp.float32)
        ones16 = jnp.ones((16,), jnp.float32)

        @pl.loop(0, _NPAD, step=16)
        def _(i):
            deg_v[pl.ds(i, 16)] = zero16

        def count_rows(n):
            @pl.loop(0, n)
            def _(r):
                for kk in range(128 // 16):
                    idx = dst_v[r, pl.ds(kk * 16, 16)]
                    plsc.addupdate_scatter(deg_v, [idx], ones16)

        @pl.when(wid < _NW - 1)
        def _():
            pltpu.sync_copy(ei_hbm.at[1, pl.ds(wid * _RPW, _RPW)], dst_v)
            count_rows(_RPW)

        @pl.when(wid == _NW - 1)
        def _():
            pltpu.sync_copy(tail_hbm.at[1], dst_v.at[pl.ds(0, _TAIL)])
            count_rows(_TAIL)

        pltpu.sync_copy(deg_v, out_hbm.at[wid])

    return k(ei3, tail3)


_RPW = 80
_ROWS = _E // 128
_TAIL = _ROWS - 31 * _RPW


def _sc_aggregate(y, ei3, tail3, z):
    @functools.partial(
        pl.kernel,
        out_type=jax.ShapeDtypeStruct((_NC, _NPAD, _D), jnp.float32),
        mesh=_mesh(),
        scratch_types=[
            pltpu.VMEM((_RPW // 2, 128), jnp.int32),
            pltpu.VMEM((_RPW // 2, 128), jnp.int32),
            pltpu.VMEM((128, _D), jnp.float32),
            pltpu.VMEM((128, _D), jnp.float32),
            pltpu.VMEM_SHARED((_NPAD, _D), jnp.float32),
            pltpu.SemaphoreType.DMA,
            pltpu.SemaphoreType.DMA,
            pltpu.SemaphoreType.DMA,
            pltpu.SemaphoreType.DMA,
        ],
    )
    def k(y_hbm, ei_hbm, tail_hbm, z_hbm, out_hbm, src_v, dst_v, b0, b1,
          acc_sh, gs0, gs1, ss0, ss1):
        c = lax.axis_index("c")
        s = lax.axis_index("s")
        wid = s * _NC + c
        rps = _NPAD // _NS
        base = wid * _RPW
        half = _RPW // 2

        pltpu.sync_copy(z_hbm, acc_sh.at[pl.ds(s * rps, rps)])
        plsc.subcore_barrier()

        def g_start(j, buf, sem):
            pltpu.async_copy(y_hbm.at[src_v.at[j]], buf, sem)

        def g_wait(j, buf, sem):
            pltpu.make_async_copy(y_hbm.at[src_v.at[j]], buf, sem).wait()

        def scat(j, buf):
            pltpu.sync_copy(buf, acc_sh.at[dst_v.at[j]], add=True)

        def pipeline(n):
            g_start(0, b0, gs0)

            @pl.loop(0, n - 2, step=2)
            def _(j):
                g_start(j + 1, b1, gs1)
                g_wait(j, b0, gs0)
                scat(j, b0)
                g_start(j + 2, b0, gs0)
                g_wait(j + 1, b1, gs1)
                scat(j + 1, b1)

            g_start(n - 1, b1, gs1)
            g_wait(n - 2, b0, gs0)
            scat(n - 2, b0)
            g_wait(n - 1, b1, gs1)
            scat(n - 1, b1)

        @pl.when(wid < _NW - 1)
        def _():
            @pl.loop(0, 2)
            def _(p):
                row0 = base + p * half
                pltpu.sync_copy(ei_hbm.at[0, pl.ds(row0, half)], src_v)
                pltpu.sync_copy(ei_hbm.at[1, pl.ds(row0, half)], dst_v)
                pipeline(half)

        @pl.when(wid == _NW - 1)
        def _():
            pltpu.sync_copy(tail_hbm.at[0], src_v.at[pl.ds(0, _TAIL)])
            pltpu.sync_copy(tail_hbm.at[1], dst_v.at[pl.ds(0, _TAIL)])
            pipeline(_TAIL)

        plsc.subcore_barrier()
        pltpu.sync_copy(acc_sh.at[pl.ds(s * rps, rps)],
                        out_hbm.at[c, pl.ds(s * rps, rps)])

    return k(y, ei3, tail3, z)


def _dis_of(degp_blk):
    deg = jnp.sum(degp_blk, axis=0) + 1.0
    return lax.rsqrt(deg)


def _mm_scale_body(x_ref, w_ref, degp_ref, o_ref):
    xw = jnp.dot(x_ref[...], w_ref[...], precision=_HI,
                 preferred_element_type=jnp.float32)
    dis = _dis_of(degp_ref[...])
    o_ref[...] = xw * dis[:, None]


def _mm_scale(x, w, degp):
    return pl.pallas_call(
        _mm_scale_body,
        grid=(_NB,),
        in_specs=[
            pl.BlockSpec((_BLK, _D), lambda i: (i, 0)),
            pl.BlockSpec((_D, _D), lambda i: (0, 0)),
            pl.BlockSpec((_NW, _BLK), lambda i: (0, i)),
        ],
        out_specs=pl.BlockSpec((_BLK, _D), lambda i: (i, 0)),
        out_shape=jax.ShapeDtypeStruct((_NPAD, _D), jnp.float32),
    )(x, w, degp)


def _layer2_body(p_ref, y_ref, degp_ref, b1_ref, w2_ref, o_ref):
    dis = _dis_of(degp_ref[...])
    h = dis[:, None] * (p_ref[0] + p_ref[1] + y_ref[...]) + b1_ref[...]
    h = jnp.maximum(h, 0.0)
    y2 = jnp.dot(h, w2_ref[...], precision=_HI,
                 preferred_element_type=jnp.float32)
    o_ref[...] = y2 * dis[:, None]


def _layer2(p, y1, degp, b1, w2):
    return pl.pallas_call(
        _layer2_body,
        grid=(_NB,),
        in_specs=[
            pl.BlockSpec((_NC, _BLK, _D), lambda i: (0, i, 0)),
            pl.BlockSpec((_BLK, _D), lambda i: (i, 0)),
            pl.BlockSpec((_NW, _BLK), lambda i: (0, i)),
            pl.BlockSpec((1, _D), lambda i: (0, 0)),
            pl.BlockSpec((_D, _D), lambda i: (0, 0)),
        ],
        out_specs=pl.BlockSpec((_BLK, _D), lambda i: (i, 0)),
        out_shape=jax.ShapeDtypeStruct((_NPAD, _D), jnp.float32),
    )(p, y1, degp, b1, w2)


def _pool_body(q_ref, y2_ref, degp_ref, batch_ref, b2_ref, o_ref, cnt_ref):
    i = pl.program_id(0)
    dis = _dis_of(degp_ref[...])
    nodes = dis[:, None] * (q_ref[0] + q_ref[1] + y2_ref[...])
    gids = lax.broadcasted_iota(jnp.int32, (_BLK, _G), 1)
    onehot = (batch_ref[0, 0, :][:, None] == gids).astype(jnp.float32)
    sums = lax.dot_general(onehot, nodes, (((0,), (0,)), ((), ())),
                           precision=_HI,
                           preferred_element_type=jnp.float32)
    cnts = jnp.sum(onehot, axis=0)

    @pl.when(i == 0)
    def _():
        o_ref[...] = jnp.zeros_like(o_ref)
        cnt_ref[...] = jnp.zeros_like(cnt_ref)

    o_ref[...] += sums
    cnt_ref[...] += cnts[:, None]

    @pl.when(i == _NB - 1)
    def _():
        o_ref[...] = o_ref[...] / jnp.maximum(cnt_ref[...], 1.0) + b2_ref[...]


def _pool(q, y2, degp, batch3, b2):
    return pl.pallas_call(
        _pool_body,
        grid=(_NB,),
        in_specs=[
            pl.BlockSpec((_NC, _BLK, _D), lambda i: (0, i, 0)),
            pl.BlockSpec((_BLK, _D), lambda i: (i, 0)),
            pl.BlockSpec((_NW, _BLK), lambda i: (0, i)),
            pl.BlockSpec((1, 1, _BLK), lambda i: (i, 0, 0)),
            pl.BlockSpec((1, _D), lambda i: (0, 0)),
        ],
        out_specs=pl.BlockSpec((_G, _D), lambda i: (0, 0)),
        out_shape=jax.ShapeDtypeStruct((_G, _D), jnp.float32),
        scratch_shapes=[pltpu.VMEM((_G, _D), jnp.float32)],
    )(q, y2, degp, batch3, b2)


def kernel(x, edge_index, batch, W1, b1, W2, b2):
    ei3 = edge_index.reshape(2, _ROWS, 128)
    tail3 = ei3[:, 31 * _RPW:, :]
    z = jnp.zeros((_NPAD // _NS, _D), jnp.float32)
    b1r = b1.reshape(1, _D)
    b2r = b2.reshape(1, _D)
    xp = jnp.pad(x, ((0, _NPAD - _N), (0, 0)))
    batch3 = jnp.pad(batch, (0, _NPAD - _N),
                     constant_values=_G).reshape(_NB, 1, _BLK)

    degp = _sc_degree(ei3, tail3)
    y1 = _mm_scale(xp, W1, degp)
    p = _sc_aggregate(y1, ei3, tail3, z)
    y2 = _layer2(p, y1, degp, b1r, W2)
    q = _sc_aggregate(y2, ei3, tail3, z)
    return _pool(q, y2, degp, batch3, b2r)


# speedup vs baseline: 1.3151x; 1.0063x over previous
import dataclasses
import functools

import jax
import jax.numpy as jnp
from jax import lax
from jax.experimental import pallas as pl
from jax.experimental.pallas import tpu as pltpu
from jax.experimental.pallas import tpu_sc as plsc

_N = 10000
_E = 320000
_D = 128
_G = 64
_NC, _NS = 2, 16
_NW = _NC * _NS
_NPAD = 10240
_EPW = _E // _NW
_K = 80
_CPW = _EPW // _K
_BLK = 2560
_NB = _NPAD // _BLK
_HI = lax.Precision.HIGHEST

def _mesh():
    return plsc.VectorSubcoreMesh(core_axis_name="c", subcore_axis_name="s",
                                  num_cores=_NC, num_subcores=_NS)


def _sc_params():
    cp = pltpu.CompilerParams()
    if "needs_layout_passes" in pltpu.CompilerParams.__dataclass_fields__:
        cp = dataclasses.replace(cp, needs_layout_passes=False)
    return cp


def _sc_degree(ei3, tail3):
    @functools.partial(
        pl.kernel,
        out_type=jax.ShapeDtypeStruct((_NW, _NPAD), jnp.float32),
        mesh=_mesh(),
        scratch_types=[
            pltpu.VMEM((_RPW, 128), jnp.int32),
            pltpu.VMEM((_NPAD,), jnp.float32),
        ],
        compiler_params=_sc_params(),
    )
    def k(ei_hbm, tail_hbm, out_hbm, dst_v, deg_v):
        c = lax.axis_index("c")
        s = lax.axis_index("s")
        wid = s * _NC + c
        zero16 = jnp.zeros((16,), jnp.float32)
        ones16 = jnp.ones((16,), jnp.float32)

        @pl.loop(0, _NPAD, step=16)
        def _(i):
            deg_v[pl.ds(i, 16)] = zero16

        def count_rows(n):
            @pl.loop(0, n)
            def _(r):
                for kk in range(128 // 16):
                    idx = dst_v[r, pl.ds(kk * 16, 16)]
                    plsc.addupdate_scatter(deg_v, [idx], ones16)

        @pl.when(wid < _NW - 1)
        def _():
            pltpu.sync_copy(ei_hbm.at[1, pl.ds(wid * _RPW, _RPW)], dst_v)
            count_rows(_RPW)

        @pl.when(wid == _NW - 1)
        def _():
            pltpu.sync_copy(tail_hbm.at[1], dst_v.at[pl.ds(0, _TAIL)])
            count_rows(_TAIL)

        pltpu.sync_copy(deg_v, out_hbm.at[wid])

    return k(ei3, tail3)


_RPW = 80
_ROWS = _E // 128
_TAIL = _ROWS - 31 * _RPW


def _sc_aggregate(y, ei3, tail3, z):
    @functools.partial(
        pl.kernel,
        out_type=jax.ShapeDtypeStruct((_NC, _NPAD, _D), jnp.float32),
        mesh=_mesh(),
        scratch_types=[
            pltpu.VMEM((_RPW // 2, 128), jnp.int32),
            pltpu.VMEM((_RPW // 2, 128), jnp.int32),
            pltpu.VMEM((128, _D), jnp.float32),
            pltpu.VMEM((128, _D), jnp.float32),
            pltpu.VMEM_SHARED((_NPAD, _D), jnp.float32),
            pltpu.SemaphoreType.DMA,
            pltpu.SemaphoreType.DMA,
            pltpu.SemaphoreType.DMA,
            pltpu.SemaphoreType.DMA,
        ],
    )
    def k(y_hbm, ei_hbm, tail_hbm, z_hbm, out_hbm, src_v, dst_v, b0, b1,
          acc_sh, gs0, gs1, ss0, ss1):
        c = lax.axis_index("c")
        s = lax.axis_index("s")
        wid = s * _NC + c
        rps = _NPAD // _NS
        base = wid * _RPW
        half = _RPW // 2

        pltpu.sync_copy(z_hbm, acc_sh.at[pl.ds(s * rps, rps)])
        plsc.subcore_barrier()

        def g_start(j, buf, sem):
            pltpu.async_copy(y_hbm.at[src_v.at[j]], buf, sem)

        def g_wait(j, buf, sem):
            pltpu.make_async_copy(y_hbm.at[src_v.at[j]], buf, sem).wait()

        def scat(j, buf):
            pltpu.sync_copy(buf, acc_sh.at[dst_v.at[j]], add=True)

        def pipeline(n):
            g_start(0, b0, gs0)

            @pl.loop(0, n - 2, step=2)
            def _(j):
                g_start(j + 1, b1, gs1)
                g_wait(j, b0, gs0)
                scat(j, b0)
                g_start(j + 2, b0, gs0)
                g_wait(j + 1, b1, gs1)
                scat(j + 1, b1)

            g_start(n - 1, b1, gs1)
            g_wait(n - 2, b0, gs0)
            scat(n - 2, b0)
            g_wait(n - 1, b1, gs1)
            scat(n - 1, b1)

        @pl.when(wid < _NW - 1)
        def _():
            @pl.loop(0, 2)
            def _(p):
                row0 = base + p * half
                pltpu.sync_copy(ei_hbm.at[0, pl.ds(row0, half)], src_v)
                pltpu.sync_copy(ei_hbm.at[1, pl.ds(row0, half)], dst_v)
                pipeline(half)

        @pl.when(wid == _NW - 1)
        def _():
            pltpu.sync_copy(tail_hbm.at[0], src_v.at[pl.ds(0, _TAIL)])
            pltpu.sync_copy(tail_hbm.at[1], dst_v.at[pl.ds(0, _TAIL)])
            pipeline(_TAIL)

        plsc.subcore_barrier()
        pltpu.sync_copy(acc_sh.at[pl.ds(s * rps, rps)],
                        out_hbm.at[c, pl.ds(s * rps, rps)])

    return k(y, ei3, tail3, z)


def _dis_of(degp_blk):
    deg = jnp.sum(degp_blk, axis=0) + 1.0
    return lax.rsqrt(deg)


def _mm_scale_body(x_ref, w_ref, degp_ref, o_ref):
    xw = jnp.dot(x_ref[...], w_ref[...], precision=_HI,
                 preferred_element_type=jnp.float32)
    dis = _dis_of(degp_ref[...])
    o_ref[...] = xw * dis[:, None]


def _mm_scale(x, w, degp):
    return pl.pallas_call(
        _mm_scale_body,
        grid=(_NB,),
        in_specs=[
            pl.BlockSpec((_BLK, _D), lambda i: (i, 0)),
            pl.BlockSpec((_D, _D), lambda i: (0, 0)),
            pl.BlockSpec((_NW, _BLK), lambda i: (0, i)),
        ],
        out_specs=pl.BlockSpec((_BLK, _D), lambda i: (i, 0)),
        out_shape=jax.ShapeDtypeStruct((_NPAD, _D), jnp.float32),
    )(x, w, degp)


def _layer2_body(p_ref, y_ref, degp_ref, b1_ref, w2_ref, o_ref):
    dis = _dis_of(degp_ref[...])
    h = dis[:, None] * (p_ref[0] + p_ref[1] + y_ref[...]) + b1_ref[...]
    h = jnp.maximum(h, 0.0)
    y2 = jnp.dot(h, w2_ref[...], precision=_HI,
                 preferred_element_type=jnp.float32)
    o_ref[...] = y2 * dis[:, None]


def _layer2(p, y1, degp, b1, w2):
    return pl.pallas_call(
        _layer2_body,
        grid=(_NB,),
        in_specs=[
            pl.BlockSpec((_NC, _BLK, _D), lambda i: (0, i, 0)),
            pl.BlockSpec((_BLK, _D), lambda i: (i, 0)),
            pl.BlockSpec((_NW, _BLK), lambda i: (0, i)),
            pl.BlockSpec((1, _D), lambda i: (0, 0)),
            pl.BlockSpec((_D, _D), lambda i: (0, 0)),
        ],
        out_specs=pl.BlockSpec((_BLK, _D), lambda i: (i, 0)),
        out_shape=jax.ShapeDtypeStruct((_NPAD, _D), jnp.float32),
    )(p, y1, degp, b1, w2)


def _pool_body(q_ref, y2_ref, degp_ref, batch_ref, b2_ref, o_ref, cnt_ref):
    i = pl.program_id(0)
    dis = _dis_of(degp_ref[...])
    nodes = dis[:, None] * (q_ref[0] + q_ref[1] + y2_ref[...])
    gids = lax.broadcasted_iota(jnp.int32, (_BLK, _G), 1)
    onehot = (batch_ref[0, 0, :][:, None] == gids).astype(jnp.float32)
    sums = lax.dot_general(onehot, nodes, (((0,), (0,)), ((), ())),
                           precision=_HI,
                           preferred_element_type=jnp.float32)
    cnts = jnp.sum(onehot, axis=0)

    @pl.when(i == 0)
    def _():
        o_ref[...] = jnp.zeros_like(o_ref)
        cnt_ref[...] = jnp.zeros_like(cnt_ref)

    o_ref[...] += sums
    cnt_ref[...] += cnts[:, None]

    @pl.when(i == _NB - 1)
    def _():
        o_ref[...] = o_ref[...] / jnp.maximum(cnt_ref[...], 1.0) + b2_ref[...]


def _pool(q, y2, degp, batch3, b2):
    return pl.pallas_call(
        _pool_body,
        grid=(_NB,),
        in_specs=[
            pl.BlockSpec((_NC, _BLK, _D), lambda i: (0, i, 0)),
            pl.BlockSpec((_BLK, _D), lambda i: (i, 0)),
            pl.BlockSpec((_NW, _BLK), lambda i: (0, i)),
            pl.BlockSpec((1, 1, _BLK), lambda i: (i, 0, 0)),
            pl.BlockSpec((1, _D), lambda i: (0, 0)),
        ],
        out_specs=pl.BlockSpec((_G, _D), lambda i: (0, 0)),
        out_shape=jax.ShapeDtypeStruct((_G, _D), jnp.float32),
        scratch_shapes=[pltpu.VMEM((_G, _D), jnp.float32)],
    )(q, y2, degp, batch3, b2)


def kernel(x, edge_index, batch, W1, b1, W2, b2):
    ei3 = edge_index.reshape(2, _ROWS, 128)
    tail3 = ei3[:, 31 * _RPW:, :]
    z = jnp.zeros((_NPAD // _NS, _D), jnp.float32)
    b1r = b1.reshape(1, _D)
    b2r = b2.reshape(1, _D)
    xp = jnp.pad(x, ((0, _NPAD - _N), (0, 0)))
    batch3 = jnp.pad(batch, (0, _NPAD - _N),
                     constant_values=_G).reshape(_NB, 1, _BLK)

    degp = _sc_degree(ei3, tail3)
    y1 = _mm_scale(xp, W1, degp)
    p = _sc_aggregate(y1, ei3, tail3, z)
    y2 = _layer2(p, y1, degp, b1r, W2)
    q = _sc_aggregate(y2, ei3, tail3, z)
    return _pool(q, y2, degp, batch3, b2r)


# speedup vs baseline: 1.3211x; 1.0046x over previous
import dataclasses
import functools

import jax
import jax.numpy as jnp
from jax import lax
from jax.experimental import pallas as pl
from jax.experimental.pallas import tpu as pltpu
from jax.experimental.pallas import tpu_sc as plsc

_N = 10000
_E = 320000
_D = 128
_G = 64
_NC, _NS = 2, 16
_NW = _NC * _NS
_NPAD = 10240
_EPW = _E // _NW
_K = 80
_CPW = _EPW // _K
_BLK = 2560
_NB = _NPAD // _BLK
_HI = lax.Precision.HIGHEST

def _mesh():
    return plsc.VectorSubcoreMesh(core_axis_name="c", subcore_axis_name="s",
                                  num_cores=_NC, num_subcores=_NS)


def _sc_params():
    cp = pltpu.CompilerParams()
    if "needs_layout_passes" in pltpu.CompilerParams.__dataclass_fields__:
        cp = dataclasses.replace(cp, needs_layout_passes=False)
    return cp


def _sc_degree(ei3, tail3, z1):
    @functools.partial(
        pl.kernel,
        out_type=jax.ShapeDtypeStruct((_NW, _NPAD), jnp.float32),
        mesh=_mesh(),
        scratch_types=[
            pltpu.VMEM((_RPW, 128), jnp.int32),
            pltpu.VMEM((_NPAD,), jnp.float32),
        ],
        compiler_params=_sc_params(),
    )
    def k(ei_hbm, tail_hbm, z1_hbm, out_hbm, dst_v, deg_v):
        c = lax.axis_index("c")
        s = lax.axis_index("s")
        wid = s * _NC + c
        ones16 = jnp.ones((16,), jnp.float32)

        pltpu.sync_copy(z1_hbm, deg_v)

        def count_rows(n):
            @plsc.parallel_loop(0, n, 1, unroll=2)
            def _(r):
                for kk in range(128 // 16):
                    idx = dst_v[r, pl.ds(kk * 16, 16)]
                    plsc.addupdate_scatter(deg_v, [idx], ones16)

        @pl.when(wid < _NW - 1)
        def _():
            pltpu.sync_copy(ei_hbm.at[1, pl.ds(wid * _RPW, _RPW)], dst_v)
            count_rows(_RPW)

        @pl.when(wid == _NW - 1)
        def _():
            pltpu.sync_copy(tail_hbm.at[1], dst_v.at[pl.ds(0, _TAIL)])
            count_rows(_TAIL)

        pltpu.sync_copy(deg_v, out_hbm.at[wid])

    return k(ei3, tail3, z1)


_RPW = 80
_ROWS = _E // 128
_TAIL = _ROWS - 31 * _RPW


def _sc_aggregate(y, ei3, tail3, z):
    @functools.partial(
        pl.kernel,
        out_type=jax.ShapeDtypeStruct((_NC, _NPAD, _D), jnp.float32),
        mesh=_mesh(),
        scratch_types=[
            pltpu.VMEM((_RPW // 2, 128), jnp.int32),
            pltpu.VMEM((_RPW // 2, 128), jnp.int32),
            pltpu.VMEM((128, _D), jnp.float32),
            pltpu.VMEM((128, _D), jnp.float32),
            pltpu.VMEM_SHARED((_NPAD, _D), jnp.float32),
            pltpu.SemaphoreType.DMA,
            pltpu.SemaphoreType.DMA,
            pltpu.SemaphoreType.DMA,
            pltpu.SemaphoreType.DMA,
        ],
    )
    def k(y_hbm, ei_hbm, tail_hbm, z_hbm, out_hbm, src_v, dst_v, b0, b1,
          acc_sh, gs0, gs1, ss0, ss1):
        c = lax.axis_index("c")
        s = lax.axis_index("s")
        wid = s * _NC + c
        rps = _NPAD // _NS
        base = wid * _RPW
        half = _RPW // 2

        pltpu.sync_copy(z_hbm, acc_sh.at[pl.ds(s * rps, rps)])
        plsc.subcore_barrier()

        def g_start(j, buf, sem):
            pltpu.async_copy(y_hbm.at[src_v.at[j]], buf, sem)

        def g_wait(j, buf, sem):
            pltpu.make_async_copy(y_hbm.at[src_v.at[j]], buf, sem).wait()

        def scat(j, buf):
            pltpu.sync_copy(buf, acc_sh.at[dst_v.at[j]], add=True)

        def pipeline(n):
            g_start(0, b0, gs0)

            @pl.loop(0, n - 2, step=2)
            def _(j):
                g_start(j + 1, b1, gs1)
                g_wait(j, b0, gs0)
                scat(j, b0)
                g_start(j + 2, b0, gs0)
                g_wait(j + 1, b1, gs1)
                scat(j + 1, b1)

            g_start(n - 1, b1, gs1)
            g_wait(n - 2, b0, gs0)
            scat(n - 2, b0)
            g_wait(n - 1, b1, gs1)
            scat(n - 1, b1)

        @pl.when(wid < _NW - 1)
        def _():
            @pl.loop(0, 2)
            def _(p):
                row0 = base + p * half
                pltpu.sync_copy(ei_hbm.at[0, pl.ds(row0, half)], src_v)
                pltpu.sync_copy(ei_hbm.at[1, pl.ds(row0, half)], dst_v)
                pipeline(half)

        @pl.when(wid == _NW - 1)
        def _():
            pltpu.sync_copy(tail_hbm.at[0], src_v.at[pl.ds(0, _TAIL)])
            pltpu.sync_copy(tail_hbm.at[1], dst_v.at[pl.ds(0, _TAIL)])
            pipeline(_TAIL)

        plsc.subcore_barrier()
        pltpu.sync_copy(acc_sh.at[pl.ds(s * rps, rps)],
                        out_hbm.at[c, pl.ds(s * rps, rps)])

    return k(y, ei3, tail3, z)


def _dis_of(degp_blk):
    deg = jnp.sum(degp_blk, axis=0) + 1.0
    return lax.rsqrt(deg)


def _mm_scale_body(x_ref, w_ref, degp_ref, o_ref):
    xw = jnp.dot(x_ref[...], w_ref[...], precision=_HI,
                 preferred_element_type=jnp.float32)
    dis = _dis_of(degp_ref[...])
    o_ref[...] = xw * dis[:, None]


def _mm_scale(x, w, degp):
    return pl.pallas_call(
        _mm_scale_body,
        grid=(_NB,),
        in_specs=[
            pl.BlockSpec((_BLK, _D), lambda i: (i, 0)),
            pl.BlockSpec((_D, _D), lambda i: (0, 0)),
            pl.BlockSpec((_NW, _BLK), lambda i: (0, i)),
        ],
        out_specs=pl.BlockSpec((_BLK, _D), lambda i: (i, 0)),
        out_shape=jax.ShapeDtypeStruct((_NPAD, _D), jnp.float32),
    )(x, w, degp)


def _layer2_body(p_ref, y_ref, degp_ref, b1_ref, w2_ref, o_ref):
    dis = _dis_of(degp_ref[...])
    h = dis[:, None] * (p_ref[0] + p_ref[1] + y_ref[...]) + b1_ref[...]
    h = jnp.maximum(h, 0.0)
    y2 = jnp.dot(h, w2_ref[...], precision=_HI,
                 preferred_element_type=jnp.float32)
    o_ref[...] = y2 * dis[:, None]


def _layer2(p, y1, degp, b1, w2):
    return pl.pallas_call(
        _layer2_body,
        grid=(_NB,),
        in_specs=[
            pl.BlockSpec((_NC, _BLK, _D), lambda i: (0, i, 0)),
            pl.BlockSpec((_BLK, _D), lambda i: (i, 0)),
            pl.BlockSpec((_NW, _BLK), lambda i: (0, i)),
            pl.BlockSpec((1, _D), lambda i: (0, 0)),
            pl.BlockSpec((_D, _D), lambda i: (0, 0)),
        ],
        out_specs=pl.BlockSpec((_BLK, _D), lambda i: (i, 0)),
        out_shape=jax.ShapeDtypeStruct((_NPAD, _D), jnp.float32),
    )(p, y1, degp, b1, w2)


def _pool_body(q_ref, y2_ref, degp_ref, batch_ref, b2_ref, o_ref, cnt_ref):
    i = pl.program_id(0)
    dis = _dis_of(degp_ref[...])
    nodes = dis[:, None] * (q_ref[0] + q_ref[1] + y2_ref[...])
    gids = lax.broadcasted_iota(jnp.int32, (_BLK, _G), 1)
    onehot = (batch_ref[0, 0, :][:, None] == gids).astype(jnp.float32)
    sums = lax.dot_general(onehot, nodes, (((0,), (0,)), ((), ())),
                           precision=_HI,
                           preferred_element_type=jnp.float32)
    cnts = jnp.sum(onehot, axis=0)

    @pl.when(i == 0)
    def _():
        o_ref[...] = jnp.zeros_like(o_ref)
        cnt_ref[...] = jnp.zeros_like(cnt_ref)

    o_ref[...] += sums
    cnt_ref[...] += cnts[:, None]

    @pl.when(i == _NB - 1)
    def _():
        o_ref[...] = o_ref[...] / jnp.maximum(cnt_ref[...], 1.0) + b2_ref[...]


def _pool(q, y2, degp, batch3, b2):
    return pl.pallas_call(
        _pool_body,
        grid=(_NB,),
        in_specs=[
            pl.BlockSpec((_NC, _BLK, _D), lambda i: (0, i, 0)),
            pl.BlockSpec((_BLK, _D), lambda i: (i, 0)),
            pl.BlockSpec((_NW, _BLK), lambda i: (0, i)),
            pl.BlockSpec((1, 1, _BLK), lambda i: (i, 0, 0)),
            pl.BlockSpec((1, _D), lambda i: (0, 0)),
        ],
        out_specs=pl.BlockSpec((_G, _D), lambda i: (0, 0)),
        out_shape=jax.ShapeDtypeStruct((_G, _D), jnp.float32),
        scratch_shapes=[pltpu.VMEM((_G, _D), jnp.float32)],
    )(q, y2, degp, batch3, b2)


def kernel(x, edge_index, batch, W1, b1, W2, b2):
    ei3 = edge_index.reshape(2, _ROWS, 128)
    tail3 = ei3[:, 31 * _RPW:, :]
    z = jnp.zeros((_NPAD // _NS, _D), jnp.float32)
    b1r = b1.reshape(1, _D)
    b2r = b2.reshape(1, _D)
    xp = jnp.pad(x, ((0, _NPAD - _N), (0, 0)))
    batch3 = jnp.pad(batch, (0, _NPAD - _N),
                     constant_values=_G).reshape(_NB, 1, _BLK)

    z1 = jnp.zeros((_NPAD,), jnp.float32)
    degp = _sc_degree(ei3, tail3, z1)
    y1 = _mm_scale(xp, W1, degp)
    p = _sc_aggregate(y1, ei3, tail3, z)
    y2 = _layer2(p, y1, degp, b1r, W2)
    q = _sc_aggregate(y2, ei3, tail3, z)
    return _pool(q, y2, degp, batch3, b2r)


# speedup vs baseline: 1.3472x; 1.0198x over previous
import dataclasses
import functools

import jax
import jax.numpy as jnp
from jax import lax
from jax.experimental import pallas as pl
from jax.experimental.pallas import tpu as pltpu
from jax.experimental.pallas import tpu_sc as plsc

_N = 10000
_E = 320000
_D = 128
_G = 64
_NC, _NS = 2, 16
_NW = _NC * _NS
_NPAD = 10240
_EPW = _E // _NW
_K = 80
_CPW = _EPW // _K
_BLK = 2560
_NB = _NPAD // _BLK
_HI = lax.Precision.DEFAULT

def _mesh():
    return plsc.VectorSubcoreMesh(core_axis_name="c", subcore_axis_name="s",
                                  num_cores=_NC, num_subcores=_NS)


def _sc_params():
    cp = pltpu.CompilerParams()
    if "needs_layout_passes" in pltpu.CompilerParams.__dataclass_fields__:
        cp = dataclasses.replace(cp, needs_layout_passes=False)
    return cp


def _sc_degree(ei3, tail3, z1):
    @functools.partial(
        pl.kernel,
        out_type=jax.ShapeDtypeStruct((_NW, _NPAD), jnp.float32),
        mesh=_mesh(),
        scratch_types=[
            pltpu.VMEM((_RPW, 128), jnp.int32),
            pltpu.VMEM((_NPAD,), jnp.float32),
        ],
        compiler_params=_sc_params(),
    )
    def k(ei_hbm, tail_hbm, z1_hbm, out_hbm, dst_v, deg_v):
        c = lax.axis_index("c")
        s = lax.axis_index("s")
        wid = s * _NC + c
        ones16 = jnp.ones((16,), jnp.float32)

        pltpu.sync_copy(z1_hbm, deg_v)

        def count_rows(n):
            @plsc.parallel_loop(0, n, 1, unroll=2)
            def _(r):
                for kk in range(128 // 16):
                    idx = dst_v[r, pl.ds(kk * 16, 16)]
                    plsc.addupdate_scatter(deg_v, [idx], ones16)

        @pl.when(wid < _NW - 1)
        def _():
            pltpu.sync_copy(ei_hbm.at[1, pl.ds(wid * _RPW, _RPW)], dst_v)
            count_rows(_RPW)

        @pl.when(wid == _NW - 1)
        def _():
            pltpu.sync_copy(tail_hbm.at[1], dst_v.at[pl.ds(0, _TAIL)])
            count_rows(_TAIL)

        pltpu.sync_copy(deg_v, out_hbm.at[wid])

    return k(ei3, tail3, z1)


_RPW = 80
_ROWS = _E // 128
_TAIL = _ROWS - 31 * _RPW


def _sc_aggregate(y, ei3, tail3, z):
    @functools.partial(
        pl.kernel,
        out_type=jax.ShapeDtypeStruct((_NC, _NPAD, _D), jnp.float32),
        mesh=_mesh(),
        scratch_types=[
            pltpu.VMEM((_RPW // 2, 128), jnp.int32),
            pltpu.VMEM((_RPW // 2, 128), jnp.int32),
            pltpu.VMEM((128, _D), jnp.float32),
            pltpu.VMEM((128, _D), jnp.float32),
            pltpu.VMEM_SHARED((_NPAD, _D), jnp.float32),
            pltpu.SemaphoreType.DMA,
            pltpu.SemaphoreType.DMA,
            pltpu.SemaphoreType.DMA,
            pltpu.SemaphoreType.DMA,
        ],
    )
    def k(y_hbm, ei_hbm, tail_hbm, z_hbm, out_hbm, src_v, dst_v, b0, b1,
          acc_sh, gs0, gs1, ss0, ss1):
        c = lax.axis_index("c")
        s = lax.axis_index("s")
        wid = s * _NC + c
        rps = _NPAD // _NS
        base = wid * _RPW
        half = _RPW // 2

        pltpu.sync_copy(z_hbm, acc_sh.at[pl.ds(s * rps, rps)])
        plsc.subcore_barrier()

        def g_start(j, buf, sem):
            pltpu.async_copy(y_hbm.at[src_v.at[j]], buf, sem)

        def g_wait(j, buf, sem):
            pltpu.make_async_copy(y_hbm.at[src_v.at[j]], buf, sem).wait()

        def scat(j, buf):
            pltpu.sync_copy(buf, acc_sh.at[dst_v.at[j]], add=True)

        def pipeline(n):
            g_start(0, b0, gs0)

            @pl.loop(0, n - 2, step=2)
            def _(j):
                g_start(j + 1, b1, gs1)
                g_wait(j, b0, gs0)
                scat(j, b0)
                g_start(j + 2, b0, gs0)
                g_wait(j + 1, b1, gs1)
                scat(j + 1, b1)

            g_start(n - 1, b1, gs1)
            g_wait(n - 2, b0, gs0)
            scat(n - 2, b0)
            g_wait(n - 1, b1, gs1)
            scat(n - 1, b1)

        @pl.when(wid < _NW - 1)
        def _():
            @pl.loop(0, 2)
            def _(p):
                row0 = base + p * half
                pltpu.sync_copy(ei_hbm.at[0, pl.ds(row0, half)], src_v)
                pltpu.sync_copy(ei_hbm.at[1, pl.ds(row0, half)], dst_v)
                pipeline(half)

        @pl.when(wid == _NW - 1)
        def _():
            pltpu.sync_copy(tail_hbm.at[0], src_v.at[pl.ds(0, _TAIL)])
            pltpu.sync_copy(tail_hbm.at[1], dst_v.at[pl.ds(0, _TAIL)])
            pipeline(_TAIL)

        plsc.subcore_barrier()
        pltpu.sync_copy(acc_sh.at[pl.ds(s * rps, rps)],
                        out_hbm.at[c, pl.ds(s * rps, rps)])

    return k(y, ei3, tail3, z)


def _dis_of(degp_blk):
    deg = jnp.sum(degp_blk, axis=0) + 1.0
    return lax.rsqrt(deg)


def _mm_scale_body(x_ref, w_ref, degp_ref, o_ref):
    xw = jnp.dot(x_ref[...], w_ref[...], precision=_HI,
                 preferred_element_type=jnp.float32)
    dis = _dis_of(degp_ref[...])
    o_ref[...] = xw * dis[:, None]


def _mm_scale(x, w, degp):
    return pl.pallas_call(
        _mm_scale_body,
        grid=(_NB,),
        in_specs=[
            pl.BlockSpec((_BLK, _D), lambda i: (i, 0)),
            pl.BlockSpec((_D, _D), lambda i: (0, 0)),
            pl.BlockSpec((_NW, _BLK), lambda i: (0, i)),
        ],
        out_specs=pl.BlockSpec((_BLK, _D), lambda i: (i, 0)),
        out_shape=jax.ShapeDtypeStruct((_NPAD, _D), jnp.float32),
    )(x, w, degp)


def _layer2_body(p_ref, y_ref, degp_ref, b1_ref, w2_ref, o_ref):
    dis = _dis_of(degp_ref[...])
    h = dis[:, None] * (p_ref[0] + p_ref[1] + y_ref[...]) + b1_ref[...]
    h = jnp.maximum(h, 0.0)
    y2 = jnp.dot(h, w2_ref[...], precision=_HI,
                 preferred_element_type=jnp.float32)
    o_ref[...] = y2 * dis[:, None]


def _layer2(p, y1, degp, b1, w2):
    return pl.pallas_call(
        _layer2_body,
        grid=(_NB,),
        in_specs=[
            pl.BlockSpec((_NC, _BLK, _D), lambda i: (0, i, 0)),
            pl.BlockSpec((_BLK, _D), lambda i: (i, 0)),
            pl.BlockSpec((_NW, _BLK), lambda i: (0, i)),
            pl.BlockSpec((1, _D), lambda i: (0, 0)),
            pl.BlockSpec((_D, _D), lambda i: (0, 0)),
        ],
        out_specs=pl.BlockSpec((_BLK, _D), lambda i: (i, 0)),
        out_shape=jax.ShapeDtypeStruct((_NPAD, _D), jnp.float32),
    )(p, y1, degp, b1, w2)


def _pool_body(q_ref, y2_ref, degp_ref, batch_ref, b2_ref, o_ref, cnt_ref):
    i = pl.program_id(0)
    dis = _dis_of(degp_ref[...])
    nodes = dis[:, None] * (q_ref[0] + q_ref[1] + y2_ref[...])
    gids = lax.broadcasted_iota(jnp.int32, (_BLK, _G), 1)
    onehot = (batch_ref[0, 0, :][:, None] == gids).astype(jnp.float32)
    sums = lax.dot_general(onehot, nodes, (((0,), (0,)), ((), ())),
                           precision=_HI,
                           preferred_element_type=jnp.float32)
    cnts = jnp.sum(onehot, axis=0)

    @pl.when(i == 0)
    def _():
        o_ref[...] = jnp.zeros_like(o_ref)
        cnt_ref[...] = jnp.zeros_like(cnt_ref)

    o_ref[...] += sums
    cnt_ref[...] += cnts[:, None]

    @pl.when(i == _NB - 1)
    def _():
        o_ref[...] = o_ref[...] / jnp.maximum(cnt_ref[...], 1.0) + b2_ref[...]


def _pool(q, y2, degp, batch3, b2):
    return pl.pallas_call(
        _pool_body,
        grid=(_NB,),
        in_specs=[
            pl.BlockSpec((_NC, _BLK, _D), lambda i: (0, i, 0)),
            pl.BlockSpec((_BLK, _D), lambda i: (i, 0)),
            pl.BlockSpec((_NW, _BLK), lambda i: (0, i)),
            pl.BlockSpec((1, 1, _BLK), lambda i: (i, 0, 0)),
            pl.BlockSpec((1, _D), lambda i: (0, 0)),
        ],
        out_specs=pl.BlockSpec((_G, _D), lambda i: (0, 0)),
        out_shape=jax.ShapeDtypeStruct((_G, _D), jnp.float32),
        scratch_shapes=[pltpu.VMEM((_G, _D), jnp.float32)],
    )(q, y2, degp, batch3, b2)


def kernel(x, edge_index, batch, W1, b1, W2, b2):
    ei3 = edge_index.reshape(2, _ROWS, 128)
    tail3 = ei3[:, 31 * _RPW:, :]
    z = jnp.zeros((_NPAD // _NS, _D), jnp.float32)
    b1r = b1.reshape(1, _D)
    b2r = b2.reshape(1, _D)
    xp = jnp.pad(x, ((0, _NPAD - _N), (0, 0)))
    batch3 = jnp.pad(batch, (0, _NPAD - _N),
                     constant_values=_G).reshape(_NB, 1, _BLK)

    z1 = jnp.zeros((_NPAD,), jnp.float32)
    degp = _sc_degree(ei3, tail3, z1)
    y1 = _mm_scale(xp, W1, degp)
    p = _sc_aggregate(y1, ei3, tail3, z)
    y2 = _layer2(p, y1, degp, b1r, W2)
    q = _sc_aggregate(y2, ei3, tail3, z)
    return _pool(q, y2, degp, batch3, b2r)


# speedup vs baseline: 1.3548x; 1.0056x over previous
import dataclasses
import functools

import jax
import jax.numpy as jnp
from jax import lax
from jax.experimental import pallas as pl
from jax.experimental.pallas import tpu as pltpu
from jax.experimental.pallas import tpu_sc as plsc

_N = 10000
_E = 320000
_D = 128
_G = 64
_NC, _NS = 2, 16
_NW = _NC * _NS
_NPAD = 10240
_EPW = _E // _NW
_K = 80
_CPW = _EPW // _K
_BLK = 5120
_NB = _NPAD // _BLK
_HI = lax.Precision.DEFAULT

def _mesh():
    return plsc.VectorSubcoreMesh(core_axis_name="c", subcore_axis_name="s",
                                  num_cores=_NC, num_subcores=_NS)


def _sc_params():
    cp = pltpu.CompilerParams()
    if "needs_layout_passes" in pltpu.CompilerParams.__dataclass_fields__:
        cp = dataclasses.replace(cp, needs_layout_passes=False)
    return cp


def _sc_degree(ei3, tail3, z1):
    @functools.partial(
        pl.kernel,
        out_type=jax.ShapeDtypeStruct((_NW, _NPAD), jnp.float32),
        mesh=_mesh(),
        scratch_types=[
            pltpu.VMEM((_RPW, 128), jnp.int32),
            pltpu.VMEM((_NPAD,), jnp.float32),
        ],
        compiler_params=_sc_params(),
    )
    def k(ei_hbm, tail_hbm, z1_hbm, out_hbm, dst_v, deg_v):
        c = lax.axis_index("c")
        s = lax.axis_index("s")
        wid = s * _NC + c
        ones16 = jnp.ones((16,), jnp.float32)

        pltpu.sync_copy(z1_hbm, deg_v)

        def count_rows(n):
            @plsc.parallel_loop(0, n, 1, unroll=2)
            def _(r):
                for kk in range(128 // 16):
                    idx = dst_v[r, pl.ds(kk * 16, 16)]
                    plsc.addupdate_scatter(deg_v, [idx], ones16)

        @pl.when(wid < _NW - 1)
        def _():
            pltpu.sync_copy(ei_hbm.at[1, pl.ds(wid * _RPW, _RPW)], dst_v)
            count_rows(_RPW)

        @pl.when(wid == _NW - 1)
        def _():
            pltpu.sync_copy(tail_hbm.at[1], dst_v.at[pl.ds(0, _TAIL)])
            count_rows(_TAIL)

        pltpu.sync_copy(deg_v, out_hbm.at[wid])

    return k(ei3, tail3, z1)


_RPW = 80
_ROWS = _E // 128
_TAIL = _ROWS - 31 * _RPW


def _sc_aggregate(y, ei3, tail3, z):
    @functools.partial(
        pl.kernel,
        out_type=jax.ShapeDtypeStruct((_NC, _NPAD, _D), jnp.float32),
        mesh=_mesh(),
        scratch_types=[
            pltpu.VMEM((_RPW // 2, 128), jnp.int32),
            pltpu.VMEM((_RPW // 2, 128), jnp.int32),
            pltpu.VMEM((128, _D), jnp.float32),
            pltpu.VMEM((128, _D), jnp.float32),
            pltpu.VMEM_SHARED((_NPAD, _D), jnp.float32),
            pltpu.SemaphoreType.DMA,
            pltpu.SemaphoreType.DMA,
            pltpu.SemaphoreType.DMA,
            pltpu.SemaphoreType.DMA,
        ],
    )
    def k(y_hbm, ei_hbm, tail_hbm, z_hbm, out_hbm, src_v, dst_v, b0, b1,
          acc_sh, gs0, gs1, ss0, ss1):
        c = lax.axis_index("c")
        s = lax.axis_index("s")
        wid = s * _NC + c
        rps = _NPAD // _NS
        base = wid * _RPW
        half = _RPW // 2

        pltpu.sync_copy(z_hbm, acc_sh.at[pl.ds(s * rps, rps)])
        plsc.subcore_barrier()

        def g_start(j, buf, sem):
            pltpu.async_copy(y_hbm.at[src_v.at[j]], buf, sem)

        def g_wait(j, buf, sem):
            pltpu.make_async_copy(y_hbm.at[src_v.at[j]], buf, sem).wait()

        def scat(j, buf):
            pltpu.sync_copy(buf, acc_sh.at[dst_v.at[j]], add=True)

        def pipeline(n):
            g_start(0, b0, gs0)

            @pl.loop(0, n - 2, step=2)
            def _(j):
                g_start(j + 1, b1, gs1)
                g_wait(j, b0, gs0)
                scat(j, b0)
                g_start(j + 2, b0, gs0)
                g_wait(j + 1, b1, gs1)
                scat(j + 1, b1)

            g_start(n - 1, b1, gs1)
            g_wait(n - 2, b0, gs0)
            scat(n - 2, b0)
            g_wait(n - 1, b1, gs1)
            scat(n - 1, b1)

        @pl.when(wid < _NW - 1)
        def _():
            @pl.loop(0, 2)
            def _(p):
                row0 = base + p * half
                pltpu.sync_copy(ei_hbm.at[0, pl.ds(row0, half)], src_v)
                pltpu.sync_copy(ei_hbm.at[1, pl.ds(row0, half)], dst_v)
                pipeline(half)

        @pl.when(wid == _NW - 1)
        def _():
            pltpu.sync_copy(tail_hbm.at[0], src_v.at[pl.ds(0, _TAIL)])
            pltpu.sync_copy(tail_hbm.at[1], dst_v.at[pl.ds(0, _TAIL)])
            pipeline(_TAIL)

        plsc.subcore_barrier()
        pltpu.sync_copy(acc_sh.at[pl.ds(s * rps, rps)],
                        out_hbm.at[c, pl.ds(s * rps, rps)])

    return k(y, ei3, tail3, z)


def _dis_of(degp_blk):
    deg = jnp.sum(degp_blk, axis=0) + 1.0
    return lax.rsqrt(deg)


def _mm_scale_body(x_ref, w_ref, degp_ref, o_ref):
    xw = jnp.dot(x_ref[...], w_ref[...], precision=_HI,
                 preferred_element_type=jnp.float32)
    dis = _dis_of(degp_ref[...])
    o_ref[...] = xw * dis[:, None]


def _mm_scale(x, w, degp):
    return pl.pallas_call(
        _mm_scale_body,
        grid=(_NB,),
        in_specs=[
            pl.BlockSpec((_BLK, _D), lambda i: (i, 0)),
            pl.BlockSpec((_D, _D), lambda i: (0, 0)),
            pl.BlockSpec((_NW, _BLK), lambda i: (0, i)),
        ],
        out_specs=pl.BlockSpec((_BLK, _D), lambda i: (i, 0)),
        out_shape=jax.ShapeDtypeStruct((_NPAD, _D), jnp.float32),
    )(x, w, degp)


def _layer2_body(p_ref, y_ref, degp_ref, b1_ref, w2_ref, o_ref):
    dis = _dis_of(degp_ref[...])
    h = dis[:, None] * (p_ref[0] + p_ref[1] + y_ref[...]) + b1_ref[...]
    h = jnp.maximum(h, 0.0)
    y2 = jnp.dot(h, w2_ref[...], precision=_HI,
                 preferred_element_type=jnp.float32)
    o_ref[...] = y2 * dis[:, None]


def _layer2(p, y1, degp, b1, w2):
    return pl.pallas_call(
        _layer2_body,
        grid=(_NB,),
        in_specs=[
            pl.BlockSpec((_NC, _BLK, _D), lambda i: (0, i, 0)),
            pl.BlockSpec((_BLK, _D), lambda i: (i, 0)),
            pl.BlockSpec((_NW, _BLK), lambda i: (0, i)),
            pl.BlockSpec((1, _D), lambda i: (0, 0)),
            pl.BlockSpec((_D, _D), lambda i: (0, 0)),
        ],
        out_specs=pl.BlockSpec((_BLK, _D), lambda i: (i, 0)),
        out_shape=jax.ShapeDtypeStruct((_NPAD, _D), jnp.float32),
    )(p, y1, degp, b1, w2)


def _pool_body(q_ref, y2_ref, degp_ref, batch_ref, b2_ref, o_ref, cnt_ref):
    i = pl.program_id(0)
    dis = _dis_of(degp_ref[...])
    nodes = dis[:, None] * (q_ref[0] + q_ref[1] + y2_ref[...])
    gids = lax.broadcasted_iota(jnp.int32, (_BLK, _G), 1)
    onehot = (batch_ref[0, 0, :][:, None] == gids).astype(jnp.float32)
    sums = lax.dot_general(onehot, nodes, (((0,), (0,)), ((), ())),
                           precision=_HI,
                           preferred_element_type=jnp.float32)
    cnts = jnp.sum(onehot, axis=0)

    @pl.when(i == 0)
    def _():
        o_ref[...] = jnp.zeros_like(o_ref)
        cnt_ref[...] = jnp.zeros_like(cnt_ref)

    o_ref[...] += sums
    cnt_ref[...] += cnts[:, None]

    @pl.when(i == _NB - 1)
    def _():
        o_ref[...] = o_ref[...] / jnp.maximum(cnt_ref[...], 1.0) + b2_ref[...]


def _pool(q, y2, degp, batch3, b2):
    return pl.pallas_call(
        _pool_body,
        grid=(_NB,),
        in_specs=[
            pl.BlockSpec((_NC, _BLK, _D), lambda i: (0, i, 0)),
            pl.BlockSpec((_BLK, _D), lambda i: (i, 0)),
            pl.BlockSpec((_NW, _BLK), lambda i: (0, i)),
            pl.BlockSpec((1, 1, _BLK), lambda i: (i, 0, 0)),
            pl.BlockSpec((1, _D), lambda i: (0, 0)),
        ],
        out_specs=pl.BlockSpec((_G, _D), lambda i: (0, 0)),
        out_shape=jax.ShapeDtypeStruct((_G, _D), jnp.float32),
        scratch_shapes=[pltpu.VMEM((_G, _D), jnp.float32)],
    )(q, y2, degp, batch3, b2)


def kernel(x, edge_index, batch, W1, b1, W2, b2):
    ei3 = edge_index.reshape(2, _ROWS, 128)
    tail3 = ei3[:, 31 * _RPW:, :]
    z = jnp.zeros((_NPAD // _NS, _D), jnp.float32)
    b1r = b1.reshape(1, _D)
    b2r = b2.reshape(1, _D)
    xp = jnp.pad(x, ((0, _NPAD - _N), (0, 0)))
    batch3 = jnp.pad(batch, (0, _NPAD - _N),
                     constant_values=_G).reshape(_NB, 1, _BLK)

    z1 = jnp.zeros((_NPAD,), jnp.float32)
    degp = _sc_degree(ei3, tail3, z1)
    y1 = _mm_scale(xp, W1, degp)
    p = _sc_aggregate(y1, ei3, tail3, z)
    y2 = _layer2(p, y1, degp, b1r, W2)
    q = _sc_aggregate(y2, ei3, tail3, z)
    return _pool(q, y2, degp, batch3, b2r)
